# baseline (speedup 1.0000x reference)
import functools

import jax
import jax.numpy as jnp
from jax import lax
from jax.experimental import pallas as pl
from jax.experimental.pallas import tpu as pltpu
from jax.experimental.pallas import tpu_sc as plsc

F32 = jnp.float32
N = 10000
E = 320000
IN = 128
D1 = 192
D2 = 320
NB_GRAPHS = 16
EPS = 1e-5

N_PAD = 10016
STRIPE = N_PAD // 16
EPW = 20480
E_PAD = EPW * 16
NIDX = 8
NEA = 4
K1, NCH1 = 128, 160
K2, NCH2 = 80, 256

BN = 1000
NB = N // BN


def _spmm_body(with_ea, fc, kk, nch, nrow, gl, il, *refs):
    d = nrow - gl
    if with_ea:
        tbl, sd, eap, zfc, zea, out, eaout = refs[:7]
        scr = refs[7:]
    else:
        tbl, sd, zfc, out = refs[:4]
        scr = refs[4:]
    idxb = scr[:NIDX]
    rows = scr[NIDX:NIDX + nrow]
    k = NIDX + nrow
    if with_ea:
        earows = scr[k:k + NEA]
        k += NEA
    acc = scr[k]
    k += 1
    if with_ea:
        eacc = scr[k]
        k += 1
    isem = scr[k:k + NIDX]
    gsem = scr[k + NIDX:k + NIDX + nrow]
    ssem = scr[k + NIDX + nrow:k + NIDX + 2 * nrow]
    k += NIDX + 2 * nrow
    if with_ea:
        elsem = scr[k:k + NEA]
        essem = scr[k + NEA:k + 2 * NEA]

    c = lax.axis_index("c")
    s = lax.axis_index("s")
    r0 = s * STRIPE

    def fire_idx(ch, bi):
        pltpu.async_copy(sd.at[s, ch], idxb[bi], isem[bi])

    def wait_idx(ch, bi):
        pltpu.make_async_copy(sd.at[s, ch], idxb[bi], isem[bi]).wait()

    def fire_g(bi, br):
        pltpu.async_copy(tbl.at[c].at[idxb[bi].at[0]], rows[br], gsem[br])

    def drain_g(bi, br):
        pltpu.make_async_copy(tbl.at[c].at[idxb[bi].at[0]], rows[br],
                              gsem[br]).wait()

    def fire_s(bi, br):
        pltpu.async_copy(rows[br], acc.at[idxb[bi].at[1]], ssem[br], add=True)

    def drain_s(bi, br):
        pltpu.make_async_copy(rows[br], acc.at[idxb[bi].at[1]],
                              ssem[br]).wait()

    def fire_el(ch, be):
        pltpu.async_copy(eap.at[s, ch], earows[be], elsem[be])

    def wait_el(ch, be):
        pltpu.make_async_copy(eap.at[s, ch], earows[be], elsem[be]).wait()

    def fire_es(bi, be):
        pltpu.async_copy(earows[be], eacc.at[idxb[bi].at[1]], essem[be],
                         add=True)

    def drain_es(bi, be):
        pltpu.make_async_copy(earows[be], eacc.at[idxb[bi].at[1]],
                              essem[be]).wait()

    pltpu.sync_copy(zfc.at[pl.ds(r0, STRIPE)], acc.at[pl.ds(r0, STRIPE)])
    if with_ea:
        @pl.when(c == 0)
        def _():
            pltpu.sync_copy(zea.at[pl.ds(r0, STRIPE)],
                            eacc.at[pl.ds(r0, STRIPE)])
    plsc.subcore_barrier()

    for t in range(il):
        fire_idx(t, t % NIDX)
    for t in range(gl):
        wait_idx(t, t % NIDX)
        fire_g(t % NIDX, t % nrow)
    if with_ea:
        @pl.when(c == 0)
        def _():
            fire_el(0, 0)
            fire_el(1, 1)

    def body(g, carry):
        for u in range(8):
            t = g * 8 + u
            drain_g(u % NIDX, u % nrow)
            if with_ea:
                @pl.when(c == 0)
                def _(t=t, u=u):
                    if u >= 2:
                        drain_es((u - 2) % NIDX, (u - 2) % NEA)
                    else:
                        @pl.when(g >= 1)
                        def _():
                            drain_es((u - 2) % NIDX, (u - 2) % NEA)

                    @pl.when(t + 2 < nch)
                    def _():
                        fire_el(t + 2, (u + 2) % NEA)
                    wait_el(t, u % NEA)
                    fire_es(u % NIDX, u % NEA)
            fire_s(u % NIDX, u % nrow)
            if u >= d:
                drain_s((u - d) % NIDX, (u - d) % nrow)
            else:
                @pl.when(g >= 1)
                def _(u=u):
                    drain_s((u - d) % NIDX, (u - d) % nrow)

            @pl.when(t + il < nch)
            def _(t=t, u=u):
                fire_idx(t + il, (u + il) % NIDX)

            @pl.when(t + gl < nch)
            def _(t=t, u=u):
                wait_idx(t + gl, (u + gl) % NIDX)
                fire_g((u + gl) % NIDX, (u + gl) % nrow)
        return carry

    lax.fori_loop(0, nch // 8, body, 0)
    for t in range(nch - d, nch):
        drain_s(t % NIDX, t % nrow)
    if with_ea:
        @pl.when(c == 0)
        def _():
            for t in (nch - 2, nch - 1):
                drain_es(t % NIDX, t % NEA)
    plsc.subcore_barrier()
    pltpu.sync_copy(acc.at[pl.ds(r0, STRIPE)],
                    out.at[c].at[pl.ds(r0, STRIPE)])
    if with_ea:
        @pl.when(c == 0)
        def _():
            pltpu.sync_copy(eacc.at[pl.ds(r0, STRIPE)],
                            eaout.at[pl.ds(r0, STRIPE)])


def _make_spmm(fc, kk, nch, nrow, gl, il, with_ea):
    mesh = plsc.VectorSubcoreMesh(core_axis_name="c", subcore_axis_name="s")
    out_type = [jax.ShapeDtypeStruct((2, N_PAD, fc), F32)]
    if with_ea:
        out_type.append(jax.ShapeDtypeStruct((N_PAD, 8), F32))
    scratch = [pltpu.VMEM((2, kk), jnp.int32) for _ in range(NIDX)]
    scratch += [pltpu.VMEM((kk, fc), F32) for _ in range(nrow)]
    if with_ea:
        scratch += [pltpu.VMEM((K1, 8), F32) for _ in range(NEA)]
    scratch.append(pltpu.VMEM_SHARED((N_PAD, fc), F32))
    if with_ea:
        scratch.append(pltpu.VMEM_SHARED((N_PAD, 8), F32))
    nsem = NIDX + 2 * nrow + (2 * NEA if with_ea else 0)
    scratch += [pltpu.SemaphoreType.DMA for _ in range(nsem)]
    return pl.kernel(
        functools.partial(_spmm_body, with_ea, fc, kk, nch, nrow, gl, il),
        out_type=out_type,
        mesh=mesh,
        scratch_types=scratch,
        compiler_params=pltpu.CompilerParams(use_tc_tiling_on_sc=False),
    )


def _table1_body(x_ref, w_ref, tl_ref, tr_ref):
    t = jnp.dot(x_ref[...], w_ref[...], preferred_element_type=F32)
    tl_ref[...] = t[:, : D1 // 2]
    tr_ref[...] = t[:, D1 // 2:]


def _table2_body(x_ref, w_ref, tl_ref, tr_ref):
    t = jnp.dot(x_ref[...], w_ref[...], preferred_element_type=F32)
    tl_ref[...] = t[:, : D2 // 2]
    tr_ref[...] = t[:, D2 // 2:]


def _combine1_body(agg_ref, ea_ref, deg_ref, x_ref, ws_ref, bs_ref, we_ref,
                   bm_ref, be_ref, g_ref, bt_ref, out_ref, ssum, ssq):
    p = pl.program_id(0)
    i = pl.program_id(1)
    h = (agg_ref[...]
         + jnp.dot(ea_ref[...], we_ref[...], preferred_element_type=F32)
         + deg_ref[...] * (bm_ref[...] + be_ref[...])
         + jnp.dot(x_ref[...], ws_ref[...], preferred_element_type=F32)
         + bs_ref[...])

    @pl.when(jnp.logical_and(p == 0, i == 0))
    def _():
        ssum[...] = jnp.zeros_like(ssum)
        ssq[...] = jnp.zeros_like(ssq)

    @pl.when(p == 0)
    def _():
        ssum[...] += jnp.sum(h, axis=0, keepdims=True)
        ssq[...] += jnp.sum(h * h, axis=0, keepdims=True)

    @pl.when(p == 1)
    def _():
        mu = ssum[...] * (1.0 / N)
        var = ssq[...] * (1.0 / N) - mu * mu
        out_ref[...] = (h - mu) * lax.rsqrt(var + EPS) * g_ref[...] + bt_ref[...]


def _final_body(agg_ref, ea_ref, deg_ref, x1_ref, we_ref, bm_ref, be_ref,
                g_ref, bt_ref, batch_ref, wf1_ref, bf1_ref, a_ref, wf2_ref,
                bf2_ref, out_ref, ssum, ssq, sadd_h, sadd_x, smax_h, smax_x,
                scnt):
    p = pl.program_id(0)
    i = pl.program_id(1)
    x1b = x1_ref[...]
    h = (agg_ref[...]
         + jnp.dot(ea_ref[...], we_ref[...], preferred_element_type=F32)
         + deg_ref[...] * (bm_ref[...] + be_ref[...])
         + x1b)

    @pl.when(jnp.logical_and(p == 0, i == 0))
    def _():
        ssum[...] = jnp.zeros_like(ssum)
        ssq[...] = jnp.zeros_like(ssq)

    @pl.when(p == 0)
    def _():
        ssum[...] += jnp.sum(h, axis=0, keepdims=True)
        ssq[...] += jnp.sum(h * h, axis=0, keepdims=True)

    @pl.when(p == 1)
    def _():
        mu = ssum[...] * (1.0 / N)
        var = ssq[...] * (1.0 / N) - mu * mu
        h2 = (h - mu) * lax.rsqrt(var + EPS) * g_ref[...] + bt_ref[...]

        @pl.when(i == 0)
        def _():
            sadd_h[...] = jnp.zeros_like(sadd_h)
            sadd_x[...] = jnp.zeros_like(sadd_x)
            smax_h[...] = jnp.full_like(smax_h, -jnp.inf)
            smax_x[...] = jnp.full_like(smax_x, -jnp.inf)
            scnt[...] = jnp.zeros_like(scnt)

        bids = batch_ref[...]
        onehot = (bids == lax.broadcasted_iota(jnp.int32, (BN, NB_GRAPHS), 1)
                  ).astype(F32)
        dims = (((0,), (0,)), ((), ()))
        sadd_h[...] += lax.dot_general(onehot, h2, dims, preferred_element_type=F32)
        sadd_x[...] += lax.dot_general(onehot, x1b, dims, preferred_element_type=F32)
        scnt[...] += lax.dot_general(onehot, jnp.ones((BN, 1), F32), dims,
                                     preferred_element_type=F32)
        bmin = batch_ref[0, 0]
        bmax = batch_ref[BN - 1, 0]
        for b in range(NB_GRAPHS):
            @pl.when(jnp.logical_and(b >= bmin, b <= bmax))
            def _(b=b):
                m = bids == b
                vh = jnp.where(m, h2, -jnp.inf)
                vx = jnp.where(m, x1b, -jnp.inf)
                smax_h[b:b + 1, :] = jnp.maximum(
                    smax_h[b:b + 1, :], jnp.max(vh, axis=0, keepdims=True))
                smax_x[b:b + 1, :] = jnp.maximum(
                    smax_x[b:b + 1, :], jnp.max(vx, axis=0, keepdims=True))

        @pl.when(i == NB - 1)
        def _():
            cnt = jnp.maximum(scnt[...], 1.0)
            pooled = jnp.concatenate(
                [sadd_h[...], sadd_x[...], smax_h[...], smax_x[...],
                 sadd_h[...] / cnt, sadd_x[...] / cnt], axis=1)
            hh = jnp.dot(pooled, wf1_ref[...], preferred_element_type=F32) + bf1_ref[...]
            a = a_ref[0, 0]
            hh = jnp.where(hh >= 0, hh, a * hh)
            o = jnp.dot(hh, wf2_ref[...], preferred_element_type=F32) + bf2_ref[...]
            mx = jnp.max(o, axis=1, keepdims=True)
            lse = mx + jnp.log(jnp.sum(jnp.exp(o - mx), axis=1, keepdims=True))
            out_ref[...] = o - lse


def kernel(x, edge_index, edge_attr, batch,
           W_msg1, b_msg1, W_edge1, b_edge1, W_self1, b_self1, g1, bt1,
           W_msg2, b_msg2, W_edge2, b_edge2, g2, bt2,
           W_fc1, b_fc1, prelu_a, W_fc2, b_fc2):
    src = edge_index[0].astype(jnp.int32)
    dst = edge_index[1].astype(jnp.int32)
    pad = E_PAD - E
    src_f = jnp.concatenate([src, jnp.zeros((pad,), jnp.int32)])
    dst_f = jnp.concatenate([dst, jnp.full((pad,), N, jnp.int32)])
    sd1 = jnp.stack([src_f.reshape(16, NCH1, K1),
                     dst_f.reshape(16, NCH1, K1)], axis=2)
    sd2 = jnp.stack([src_f.reshape(16, NCH2, K2),
                     dst_f.reshape(16, NCH2, K2)], axis=2)
    ea_p = jnp.concatenate(
        [jnp.concatenate([edge_attr, jnp.ones((E, 1), F32),
                          jnp.zeros((E, 3), F32)], axis=1),
         jnp.zeros((pad, 8), F32)], axis=0).reshape(16, NCH1, K1, 8)

    h1f = D1 // 2
    tl, tr = pl.pallas_call(
        _table1_body,
        grid=(NB,),
        in_specs=[pl.BlockSpec((BN, IN), lambda i: (i, 0)),
                  pl.BlockSpec((IN, D1), lambda i: (0, 0))],
        out_specs=[pl.BlockSpec((BN, h1f), lambda i: (i, 0)),
                   pl.BlockSpec((BN, h1f), lambda i: (i, 0))],
        out_shape=[jax.ShapeDtypeStruct((N, h1f), F32),
                   jax.ShapeDtypeStruct((N, h1f), F32)],
    )(x, W_msg1)
    t1 = jnp.stack([tl, tr])

    z1 = jnp.zeros((N_PAD, h1f), F32)
    zea = jnp.zeros((N_PAD, 8), F32)
    agg1s, ea_acc = _make_spmm(h1f, K1, NCH1, 4, 2, 6, True)(
        t1, sd1, ea_p, z1, zea)
    agg1 = jnp.concatenate([agg1s[0, :N], agg1s[1, :N]], axis=1)
    ea_agg = ea_acc[:N, :4]
    deg = ea_acc[:N, 4:5]

    h1 = pl.pallas_call(
        _combine1_body,
        grid=(2, NB),
        in_specs=[pl.BlockSpec((BN, D1), lambda p, i: (i, 0)),
                  pl.BlockSpec((BN, 4), lambda p, i: (i, 0)),
                  pl.BlockSpec((BN, 1), lambda p, i: (i, 0)),
                  pl.BlockSpec((BN, IN), lambda p, i: (i, 0)),
                  pl.BlockSpec((IN, D1), lambda p, i: (0, 0)),
                  pl.BlockSpec((1, D1), lambda p, i: (0, 0)),
                  pl.BlockSpec((4, D1), lambda p, i: (0, 0)),
                  pl.BlockSpec((1, D1), lambda p, i: (0, 0)),
                  pl.BlockSpec((1, D1), lambda p, i: (0, 0)),
                  pl.BlockSpec((1, D1), lambda p, i: (0, 0)),
                  pl.BlockSpec((1, D1), lambda p, i: (0, 0))],
        out_specs=pl.BlockSpec((BN, D1), lambda p, i: (i, 0)),
        out_shape=jax.ShapeDtypeStruct((N, D1), F32),
        scratch_shapes=[pltpu.VMEM((1, D1), F32), pltpu.VMEM((1, D1), F32)],
    )(agg1, ea_agg, deg, x, W_self1, b_self1.reshape(1, D1),
      W_edge1, b_msg1.reshape(1, D1), b_edge1.reshape(1, D1),
      g1.reshape(1, D1), bt1.reshape(1, D1))

    x1 = jnp.concatenate([h1, x], axis=1)

    h2f = D2 // 2
    t2l, t2r = pl.pallas_call(
        _table2_body,
        grid=(NB,),
        in_specs=[pl.BlockSpec((BN, D2), lambda i: (i, 0)),
                  pl.BlockSpec((D2, D2), lambda i: (0, 0))],
        out_specs=[pl.BlockSpec((BN, h2f), lambda i: (i, 0))] * 2,
        out_shape=[jax.ShapeDtypeStruct((N, h2f), F32)] * 2,
    )(x1, W_msg2)
    t2 = jnp.stack([t2l, t2r])

    z2 = jnp.zeros((N_PAD, h2f), F32)
    (agg2s,) = _make_spmm(h2f, K2, NCH2, 2, 1, 4, False)(t2, sd2, z2)
    agg2 = jnp.concatenate([agg2s[0, :N], agg2s[1, :N]], axis=1)

    out = pl.pallas_call(
        _final_body,
        grid=(2, NB),
        in_specs=[pl.BlockSpec((BN, D2), lambda p, i: (i, 0)),
                  pl.BlockSpec((BN, 4), lambda p, i: (i, 0)),
                  pl.BlockSpec((BN, 1), lambda p, i: (i, 0)),
                  pl.BlockSpec((BN, D2), lambda p, i: (i, 0)),
                  pl.BlockSpec((4, D2), lambda p, i: (0, 0)),
                  pl.BlockSpec((1, D2), lambda p, i: (0, 0)),
                  pl.BlockSpec((1, D2), lambda p, i: (0, 0)),
                  pl.BlockSpec((1, D2), lambda p, i: (0, 0)),
                  pl.BlockSpec((1, D2), lambda p, i: (0, 0)),
                  pl.BlockSpec((BN, 1), lambda p, i: (i, 0)),
                  pl.BlockSpec((6 * D2, 3 * D2), lambda p, i: (0, 0)),
                  pl.BlockSpec((1, 3 * D2), lambda p, i: (0, 0)),
                  pl.BlockSpec((1, 1), lambda p, i: (0, 0)),
                  pl.BlockSpec((3 * D2, 2), lambda p, i: (0, 0)),
                  pl.BlockSpec((1, 2), lambda p, i: (0, 0))],
        out_specs=pl.BlockSpec((NB_GRAPHS, 2), lambda p, i: (0, 0)),
        out_shape=jax.ShapeDtypeStruct((NB_GRAPHS, 2), F32),
        scratch_shapes=[pltpu.VMEM((1, D2), F32), pltpu.VMEM((1, D2), F32),
                        pltpu.VMEM((NB_GRAPHS, D2), F32),
                        pltpu.VMEM((NB_GRAPHS, D2), F32),
                        pltpu.VMEM((NB_GRAPHS, D2), F32),
                        pltpu.VMEM((NB_GRAPHS, D2), F32),
                        pltpu.VMEM((NB_GRAPHS, 1), F32)],
    )(agg2, ea_agg, deg, x1, W_edge2, b_msg2.reshape(1, D2),
      b_edge2.reshape(1, D2), g2.reshape(1, D2), bt2.reshape(1, D2),
      batch.reshape(N, 1).astype(jnp.int32), W_fc1, b_fc1.reshape(1, 3 * D2),
      prelu_a.reshape(1, 1), W_fc2, b_fc2.reshape(1, 2))
    return out


# speedup vs baseline: 1.3656x; 1.3656x over previous
import functools

import jax
import jax.numpy as jnp
from jax import lax
from jax.experimental import pallas as pl
from jax.experimental.pallas import tpu as pltpu
from jax.experimental.pallas import tpu_sc as plsc

F32 = jnp.float32
N = 10000
E = 320000
IN = 128
D1 = 192
D2 = 320
NB_GRAPHS = 16
EPS = 1e-5

N_PAD = 10016
STRIPE = N_PAD // 16
EPW = 20480
E_PAD = EPW * 16
NIDX = 8
NEA = 4
K1, NCH1 = 128, 160
K2, NCH2 = 80, 256

BN = 1000
NB = N // BN


def _spmm_body(with_ea, fc, kk, nch, nrow, gl, il, *refs):
    d = nrow - gl
    if with_ea:
        tbl, sd, eap, zfc, zea, out, eaout = refs[:7]
        scr = refs[7:]
    else:
        tbl, sd, zfc, out = refs[:4]
        scr = refs[4:]
    idxb = scr[:NIDX]
    rows = scr[NIDX:NIDX + nrow]
    k = NIDX + nrow
    if with_ea:
        earows = scr[k:k + NEA]
        k += NEA
    acc = scr[k]
    k += 1
    if with_ea:
        eacc = scr[k]
        k += 1
    isem = scr[k:k + NIDX]
    gsem = scr[k + NIDX:k + NIDX + nrow]
    ssem = scr[k + NIDX + nrow:k + NIDX + 2 * nrow]
    k += NIDX + 2 * nrow
    if with_ea:
        elsem = scr[k:k + NEA]
        essem = scr[k + NEA:k + 2 * NEA]

    c = lax.axis_index("c")
    s = lax.axis_index("s")
    r0 = s * STRIPE

    def fire_idx(ch, bi):
        pltpu.async_copy(sd.at[s, ch], idxb[bi], isem[bi])

    def wait_idx(ch, bi):
        pltpu.make_async_copy(sd.at[s, ch], idxb[bi], isem[bi]).wait()

    def fire_g(bi, br):
        pltpu.async_copy(tbl.at[c].at[idxb[bi].at[0]], rows[br], gsem[br])

    def drain_g(bi, br):
        pltpu.make_async_copy(tbl.at[c].at[idxb[bi].at[0]], rows[br],
                              gsem[br]).wait()

    def fire_s(bi, br):
        pltpu.async_copy(rows[br], acc.at[idxb[bi].at[1]], ssem[br], add=True)

    def drain_s(bi, br):
        pltpu.make_async_copy(rows[br], acc.at[idxb[bi].at[1]],
                              ssem[br]).wait()

    def fire_el(ch, be):
        pltpu.async_copy(eap.at[s, ch], earows[be], elsem[be])

    def wait_el(ch, be):
        pltpu.make_async_copy(eap.at[s, ch], earows[be], elsem[be]).wait()

    def fire_es(bi, be):
        pltpu.async_copy(earows[be], eacc.at[idxb[bi].at[1]], essem[be],
                         add=True)

    def drain_es(bi, be):
        pltpu.make_async_copy(earows[be], eacc.at[idxb[bi].at[1]],
                              essem[be]).wait()

    pltpu.sync_copy(zfc.at[pl.ds(r0, STRIPE)], acc.at[pl.ds(r0, STRIPE)])
    if with_ea:
        @pl.when(c == 0)
        def _():
            pltpu.sync_copy(zea.at[pl.ds(r0, STRIPE)],
                            eacc.at[pl.ds(r0, STRIPE)])
    plsc.subcore_barrier()

    for t in range(il):
        fire_idx(t, t % NIDX)
    for t in range(gl):
        wait_idx(t, t % NIDX)
        fire_g(t % NIDX, t % nrow)
    if with_ea:
        @pl.when(c == 0)
        def _():
            fire_el(0, 0)
            fire_el(1, 1)

    def body(g, carry):
        for u in range(8):
            t = g * 8 + u
            drain_g(u % NIDX, u % nrow)
            if with_ea:
                @pl.when(c == 0)
                def _(t=t, u=u):
                    if u >= 2:
                        drain_es((u - 2) % NIDX, (u - 2) % NEA)
                    else:
                        @pl.when(g >= 1)
                        def _():
                            drain_es((u - 2) % NIDX, (u - 2) % NEA)

                    @pl.when(t + 2 < nch)
                    def _():
                        fire_el(t + 2, (u + 2) % NEA)
                    wait_el(t, u % NEA)
                    fire_es(u % NIDX, u % NEA)
            fire_s(u % NIDX, u % nrow)
            if u >= d:
                drain_s((u - d) % NIDX, (u - d) % nrow)
            else:
                @pl.when(g >= 1)
                def _(u=u):
                    drain_s((u - d) % NIDX, (u - d) % nrow)

            @pl.when(t + il < nch)
            def _(t=t, u=u):
                fire_idx(t + il, (u + il) % NIDX)

            @pl.when(t + gl < nch)
            def _(t=t, u=u):
                wait_idx(t + gl, (u + gl) % NIDX)
                fire_g((u + gl) % NIDX, (u + gl) % nrow)
        return carry

    lax.fori_loop(0, nch // 8, body, 0)
    for t in range(nch - d, nch):
        drain_s(t % NIDX, t % nrow)
    if with_ea:
        @pl.when(c == 0)
        def _():
            for t in (nch - 2, nch - 1):
                drain_es(t % NIDX, t % NEA)
    plsc.subcore_barrier()
    pltpu.sync_copy(acc.at[pl.ds(r0, STRIPE)],
                    out.at[c].at[pl.ds(r0, STRIPE)])
    if with_ea:
        @pl.when(c == 0)
        def _():
            pltpu.sync_copy(eacc.at[pl.ds(r0, STRIPE)],
                            eaout.at[pl.ds(r0, STRIPE)])


def _make_spmm(fc, kk, nch, nrow, gl, il, with_ea):
    mesh = plsc.VectorSubcoreMesh(core_axis_name="c", subcore_axis_name="s")
    out_type = [jax.ShapeDtypeStruct((2, N_PAD, fc), F32)]
    if with_ea:
        out_type.append(jax.ShapeDtypeStruct((N_PAD, 8), F32))
    scratch = [pltpu.VMEM((2, kk), jnp.int32) for _ in range(NIDX)]
    scratch += [pltpu.VMEM((kk, fc), F32) for _ in range(nrow)]
    if with_ea:
        scratch += [pltpu.VMEM((K1, 8), F32) for _ in range(NEA)]
    scratch.append(pltpu.VMEM_SHARED((N_PAD, fc), F32))
    if with_ea:
        scratch.append(pltpu.VMEM_SHARED((N_PAD, 8), F32))
    nsem = NIDX + 2 * nrow + (2 * NEA if with_ea else 0)
    scratch += [pltpu.SemaphoreType.DMA for _ in range(nsem)]
    return pl.kernel(
        functools.partial(_spmm_body, with_ea, fc, kk, nch, nrow, gl, il),
        out_type=out_type,
        mesh=mesh,
        scratch_types=scratch,
        compiler_params=pltpu.CompilerParams(use_tc_tiling_on_sc=False),
    )


def _combine1_body(aggl_ref, aggr_ref, ea_ref, x_ref, wm1_ref, ws_ref, bs_ref,
                   we_ref, bm_ref, be_ref, g_ref, bt_ref,
                   x1_ref, h1l_ref, h1r_ref, hbuf, ssum, ssq):
    p = pl.program_id(0)
    i = pl.program_id(1)
    xb = x_ref[...]

    @pl.when(p == 0)
    def _():
        eab = ea_ref[...]
        aggx = jnp.concatenate([aggl_ref[0], aggr_ref[0]], axis=1)
        h = (jnp.dot(aggx, wm1_ref[...], preferred_element_type=F32)
             + jnp.dot(eab[:, :4], we_ref[...], preferred_element_type=F32)
             + eab[:, 4:5] * (bm_ref[...] + be_ref[...])
             + jnp.dot(xb, ws_ref[...], preferred_element_type=F32)
             + bs_ref[...])
        hbuf[pl.ds(i * BN, BN), :] = h

        @pl.when(i == 0)
        def _():
            ssum[...] = jnp.zeros_like(ssum)
            ssq[...] = jnp.zeros_like(ssq)
        ssum[...] += jnp.sum(h, axis=0, keepdims=True)
        ssq[...] += jnp.sum(h * h, axis=0, keepdims=True)

    @pl.when(p == 1)
    def _():
        h = hbuf[pl.ds(i * BN, BN), :]
        mu = ssum[...] * (1.0 / N)
        var = ssq[...] * (1.0 / N) - mu * mu
        h1 = (h - mu) * lax.rsqrt(var + EPS) * g_ref[...] + bt_ref[...]
        x1_ref[...] = jnp.concatenate([h1, xb], axis=1)
        h1l_ref[...] = h1[:, : D1 // 2]
        h1r_ref[...] = h1[:, D1 // 2:]


def _final_body(agghl_ref, agghr_ref, aggxl_ref, aggxr_ref, ea_ref, x1_ref,
                wm2_ref, we_ref, bm_ref, be_ref, g_ref, bt_ref, batch_ref,
                wf1_ref, bf1_ref, a_ref, wf2_ref, bf2_ref, out_ref,
                hbuf, ssum, ssq, sadd_h, sadd_x, smax_h, smax_x, scnt):
    p = pl.program_id(0)
    i = pl.program_id(1)
    x1b = x1_ref[...]

    @pl.when(p == 0)
    def _():
        eab = ea_ref[...]
        aggx1 = jnp.concatenate([agghl_ref[0], agghr_ref[0],
                                 aggxl_ref[0], aggxr_ref[0]], axis=1)
        h = (jnp.dot(aggx1, wm2_ref[...], preferred_element_type=F32)
             + jnp.dot(eab[:, :4], we_ref[...], preferred_element_type=F32)
             + eab[:, 4:5] * (bm_ref[...] + be_ref[...])
             + x1b)
        hbuf[pl.ds(i * BN, BN), :] = h

        @pl.when(i == 0)
        def _():
            ssum[...] = jnp.zeros_like(ssum)
            ssq[...] = jnp.zeros_like(ssq)
        ssum[...] += jnp.sum(h, axis=0, keepdims=True)
        ssq[...] += jnp.sum(h * h, axis=0, keepdims=True)

    @pl.when(p == 1)
    def _():
        h = hbuf[pl.ds(i * BN, BN), :]
        mu = ssum[...] * (1.0 / N)
        var = ssq[...] * (1.0 / N) - mu * mu
        h2 = (h - mu) * lax.rsqrt(var + EPS) * g_ref[...] + bt_ref[...]

        @pl.when(i == 0)
        def _():
            sadd_h[...] = jnp.zeros_like(sadd_h)
            sadd_x[...] = jnp.zeros_like(sadd_x)
            smax_h[...] = jnp.full_like(smax_h, -jnp.inf)
            smax_x[...] = jnp.full_like(smax_x, -jnp.inf)
            scnt[...] = jnp.zeros_like(scnt)

        bids = batch_ref[...]
        onehot = (bids == lax.broadcasted_iota(jnp.int32, (BN, NB_GRAPHS), 1)
                  ).astype(F32)
        dims = (((0,), (0,)), ((), ()))
        sadd_h[...] += lax.dot_general(onehot, h2, dims, preferred_element_type=F32)
        sadd_x[...] += lax.dot_general(onehot, x1b, dims, preferred_element_type=F32)
        scnt[...] += lax.dot_general(onehot, jnp.ones((BN, 1), F32), dims,
                                     preferred_element_type=F32)
        bmin = batch_ref[0, 0]
        bmax = batch_ref[BN - 1, 0]
        for b in range(NB_GRAPHS):
            @pl.when(jnp.logical_and(b >= bmin, b <= bmax))
            def _(b=b):
                m = bids == b
                vh = jnp.where(m, h2, -jnp.inf)
                vx = jnp.where(m, x1b, -jnp.inf)
                smax_h[b:b + 1, :] = jnp.maximum(
                    smax_h[b:b + 1, :], jnp.max(vh, axis=0, keepdims=True))
                smax_x[b:b + 1, :] = jnp.maximum(
                    smax_x[b:b + 1, :], jnp.max(vx, axis=0, keepdims=True))

        @pl.when(i == NB - 1)
        def _():
            cnt = jnp.maximum(scnt[...], 1.0)
            pooled = jnp.concatenate(
                [sadd_h[...], sadd_x[...], smax_h[...], smax_x[...],
                 sadd_h[...] / cnt, sadd_x[...] / cnt], axis=1)
            hh = jnp.dot(pooled, wf1_ref[...], preferred_element_type=F32) + bf1_ref[...]
            a = a_ref[0, 0]
            hh = jnp.where(hh >= 0, hh, a * hh)
            o = jnp.dot(hh, wf2_ref[...], preferred_element_type=F32) + bf2_ref[...]
            mx = jnp.max(o, axis=1, keepdims=True)
            lse = mx + jnp.log(jnp.sum(jnp.exp(o - mx), axis=1, keepdims=True))
            out_ref[...] = o - lse


def kernel(x, edge_index, edge_attr, batch,
           W_msg1, b_msg1, W_edge1, b_edge1, W_self1, b_self1, g1, bt1,
           W_msg2, b_msg2, W_edge2, b_edge2, g2, bt2,
           W_fc1, b_fc1, prelu_a, W_fc2, b_fc2):
    src = edge_index[0].astype(jnp.int32)
    dst = edge_index[1].astype(jnp.int32)
    pad = E_PAD - E
    src_f = jnp.concatenate([src, jnp.zeros((pad,), jnp.int32)])
    dst_f = jnp.concatenate([dst, jnp.full((pad,), N, jnp.int32)])
    sd1 = jnp.stack([src_f.reshape(16, NCH1, K1),
                     dst_f.reshape(16, NCH1, K1)], axis=2)
    ea_p = jnp.concatenate(
        [jnp.concatenate([edge_attr, jnp.ones((E, 1), F32),
                          jnp.zeros((E, 3), F32)], axis=1),
         jnp.zeros((pad, 8), F32)], axis=0).reshape(16, NCH1, K1, 8)

    xf = IN // 2
    xs = jnp.stack([x[:, :xf], x[:, xf:]])
    z1 = jnp.zeros((N_PAD, xf), F32)
    zea = jnp.zeros((N_PAD, 8), F32)
    aggxs, ea_acc = _make_spmm(xf, K1, NCH1, 4, 2, 6, True)(
        xs, sd1, ea_p, z1, zea)

    h1f = D1 // 2
    x1, h1l, h1r = pl.pallas_call(
        _combine1_body,
        grid=(2, NB),
        in_specs=[pl.BlockSpec((1, BN, xf), lambda p, i: (0, i, 0)),
                  pl.BlockSpec((1, BN, xf), lambda p, i: (1, i, 0)),
                  pl.BlockSpec((BN, 8), lambda p, i: (i, 0)),
                  pl.BlockSpec((BN, IN), lambda p, i: (i, 0)),
                  pl.BlockSpec((IN, D1), lambda p, i: (0, 0)),
                  pl.BlockSpec((IN, D1), lambda p, i: (0, 0)),
                  pl.BlockSpec((1, D1), lambda p, i: (0, 0)),
                  pl.BlockSpec((4, D1), lambda p, i: (0, 0)),
                  pl.BlockSpec((1, D1), lambda p, i: (0, 0)),
                  pl.BlockSpec((1, D1), lambda p, i: (0, 0)),
                  pl.BlockSpec((1, D1), lambda p, i: (0, 0)),
                  pl.BlockSpec((1, D1), lambda p, i: (0, 0))],
        out_specs=[pl.BlockSpec((BN, D2), lambda p, i: (i, 0)),
                   pl.BlockSpec((BN, h1f), lambda p, i: (i, 0)),
                   pl.BlockSpec((BN, h1f), lambda p, i: (i, 0))],
        out_shape=[jax.ShapeDtypeStruct((N, D2), F32),
                   jax.ShapeDtypeStruct((N, h1f), F32),
                   jax.ShapeDtypeStruct((N, h1f), F32)],
        scratch_shapes=[pltpu.VMEM((N, D1), F32),
                        pltpu.VMEM((1, D1), F32), pltpu.VMEM((1, D1), F32)],
    )(aggxs, aggxs, ea_acc, x, W_msg1, W_self1, b_self1.reshape(1, D1),
      W_edge1, b_msg1.reshape(1, D1), b_edge1.reshape(1, D1),
      g1.reshape(1, D1), bt1.reshape(1, D1))

    th1 = jnp.stack([h1l, h1r])

    z2 = jnp.zeros((N_PAD, h1f), F32)
    (agghs,) = _make_spmm(h1f, K1, NCH1, 4, 2, 6, False)(th1, sd1, z2)

    out = pl.pallas_call(
        _final_body,
        grid=(2, NB),
        in_specs=[pl.BlockSpec((1, BN, h1f), lambda p, i: (0, i, 0)),
                  pl.BlockSpec((1, BN, h1f), lambda p, i: (1, i, 0)),
                  pl.BlockSpec((1, BN, xf), lambda p, i: (0, i, 0)),
                  pl.BlockSpec((1, BN, xf), lambda p, i: (1, i, 0)),
                  pl.BlockSpec((BN, 8), lambda p, i: (i, 0)),
                  pl.BlockSpec((BN, D2), lambda p, i: (i, 0)),
                  pl.BlockSpec((D2, D2), lambda p, i: (0, 0)),
                  pl.BlockSpec((4, D2), lambda p, i: (0, 0)),
                  pl.BlockSpec((1, D2), lambda p, i: (0, 0)),
                  pl.BlockSpec((1, D2), lambda p, i: (0, 0)),
                  pl.BlockSpec((1, D2), lambda p, i: (0, 0)),
                  pl.BlockSpec((1, D2), lambda p, i: (0, 0)),
                  pl.BlockSpec((BN, 1), lambda p, i: (i, 0)),
                  pl.BlockSpec((6 * D2, 3 * D2), lambda p, i: (0, 0)),
                  pl.BlockSpec((1, 3 * D2), lambda p, i: (0, 0)),
                  pl.BlockSpec((1, 1), lambda p, i: (0, 0)),
                  pl.BlockSpec((3 * D2, 2), lambda p, i: (0, 0)),
                  pl.BlockSpec((1, 2), lambda p, i: (0, 0))],
        out_specs=pl.BlockSpec((NB_GRAPHS, 2), lambda p, i: (0, 0)),
        out_shape=jax.ShapeDtypeStruct((NB_GRAPHS, 2), F32),
        scratch_shapes=[pltpu.VMEM((N, D2), F32),
                        pltpu.VMEM((1, D2), F32), pltpu.VMEM((1, D2), F32),
                        pltpu.VMEM((NB_GRAPHS, D2), F32),
                        pltpu.VMEM((NB_GRAPHS, D2), F32),
                        pltpu.VMEM((NB_GRAPHS, D2), F32),
                        pltpu.VMEM((NB_GRAPHS, D2), F32),
                        pltpu.VMEM((NB_GRAPHS, 1), F32)],
    )(agghs, agghs, aggxs, aggxs, ea_acc, x1, W_msg2, W_edge2,
      b_msg2.reshape(1, D2), b_edge2.reshape(1, D2), g2.reshape(1, D2),
      bt2.reshape(1, D2), batch.reshape(N, 1).astype(jnp.int32),
      W_fc1, b_fc1.reshape(1, 3 * D2), prelu_a.reshape(1, 1),
      W_fc2, b_fc2.reshape(1, 2))
    return out
